# baseline (speedup 1.0000x reference)
import functools

import jax
import jax.numpy as jnp
from jax import lax
from jax.experimental import pallas as pl
from jax.experimental.pallas import tpu as pltpu
from jax.experimental.pallas import tpu_sc as plsc

_VOCAB = 100000
_EMBED = 64
_BATCH = 1024

_NC = 2
_NS = 16
_NW = _NC * _NS
_RPW = _EMBED // _NW
_LANES = 16

_TV = 2048
_NV = pl.cdiv(_VOCAB, _TV)
_TAIL = _VOCAB - (_NV - 1) * _TV
_NBUF = 6


def _gather_body(table_t_hbm, idx_hbm, out_hbm, idx_v, row_v, hrow_v, sem):
    wid = lax.axis_index("s") * _NC + lax.axis_index("c")
    pltpu.sync_copy(idx_hbm, idx_v)
    for k in range(_RPW):
        r = wid * _RPW + k
        pltpu.async_copy(table_t_hbm.at[r], row_v, sem).wait()
        for c in range(_BATCH // _LANES):
            idx16 = idx_v[pl.ds(c * _LANES, _LANES)]
            hrow_v[pl.ds(c * _LANES, _LANES)] = plsc.load_gather(
                row_v, [idx16]
            )
        pltpu.sync_copy(hrow_v, out_hbm.at[r])


_gather = functools.partial(
    pl.kernel,
    mesh=plsc.VectorSubcoreMesh(core_axis_name="c", subcore_axis_name="s"),
    out_type=jax.ShapeDtypeStruct((_EMBED, _BATCH), jnp.float32),
    scratch_types=[
        pltpu.VMEM((_BATCH,), jnp.int32),
        pltpu.VMEM((_VOCAB,), jnp.float32),
        pltpu.VMEM((_BATCH,), jnp.float32),
        pltpu.SemaphoreType.DMA,
    ],
    compiler_params=pltpu.CompilerParams(needs_layout_passes=False),
)(_gather_body)


def _out_copy(acc_ref, out_hbm, sem_ref, step, last=False):
    ph = lax.rem(step, _NBUF)
    rows = _TAIL if last else _TV
    return pltpu.make_async_copy(
        acc_ref.at[ph, pl.ds(0, rows)],
        out_hbm.at[pl.ds(step * _TV, rows)],
        sem_ref.at[ph],
    )


def _proj_body(hidden_t_ref, wt_ref, out_hbm, acc_ref, sem_ref):
    i = pl.program_id(0)
    ph = lax.rem(i, _NBUF)

    @pl.when(i >= _NBUF)
    def _():
        _out_copy(acc_ref, out_hbm, sem_ref, i - _NBUF).wait()

    acc_ref[ph] = lax.dot_general(
        wt_ref[...],
        hidden_t_ref[...],
        (((0,), (0,)), ((), ())),
        preferred_element_type=jnp.float32,
    )

    @pl.when(i < _NV - 1)
    def _():
        _out_copy(acc_ref, out_hbm, sem_ref, i).start()

    @pl.when(i == _NV - 1)
    def _():
        _out_copy(acc_ref, out_hbm, sem_ref, i, last=True).start()
        for k in range(_NBUF - 1):
            _out_copy(acc_ref, out_hbm, sem_ref, _NV - _NBUF + k).wait()
        _out_copy(acc_ref, out_hbm, sem_ref, _NV - 1, last=True).wait()


def kernel(input, embed_table, expand_W):
    hidden_t = _gather(embed_table.T, input)
    logits_t = pl.pallas_call(
        _proj_body,
        grid=(_NV,),
        in_specs=[
            pl.BlockSpec((_EMBED, _BATCH), lambda i: (0, 0)),
            pl.BlockSpec((_EMBED, _TV), lambda i: (0, i)),
        ],
        out_specs=pl.BlockSpec(memory_space=pl.ANY),
        out_shape=jax.ShapeDtypeStruct((_VOCAB, _BATCH), jnp.float32),
        scratch_shapes=[
            pltpu.VMEM((_NBUF, _TV, _BATCH), jnp.float32),
            pltpu.SemaphoreType.DMA((_NBUF,)),
        ],
    )(hidden_t, expand_W.T)
    return logits_t.T


# speedup vs baseline: 1.0098x; 1.0098x over previous
import functools

import jax
import jax.numpy as jnp
from jax import lax
from jax.experimental import pallas as pl
from jax.experimental.pallas import tpu as pltpu
from jax.experimental.pallas import tpu_sc as plsc

_VOCAB = 100000
_EMBED = 64
_BATCH = 1024

_NC = 2
_NS = 16
_NW = _NC * _NS
_RPW = _EMBED // _NW
_LANES = 16

_TV = 3072
_NV = pl.cdiv(_VOCAB, _TV)
_TAIL = _VOCAB - (_NV - 1) * _TV
_NBUF = 4


def _gather_body(table_t_hbm, idx_hbm, out_hbm, idx_v, row_v, hrow_v, sem):
    wid = lax.axis_index("s") * _NC + lax.axis_index("c")
    pltpu.sync_copy(idx_hbm, idx_v)
    for k in range(_RPW):
        r = wid * _RPW + k
        pltpu.async_copy(table_t_hbm.at[r], row_v, sem).wait()
        for c in range(_BATCH // _LANES):
            idx16 = idx_v[pl.ds(c * _LANES, _LANES)]
            hrow_v[pl.ds(c * _LANES, _LANES)] = plsc.load_gather(
                row_v, [idx16]
            )
        pltpu.sync_copy(hrow_v, out_hbm.at[r])


_gather = functools.partial(
    pl.kernel,
    mesh=plsc.VectorSubcoreMesh(core_axis_name="c", subcore_axis_name="s"),
    out_type=jax.ShapeDtypeStruct((_EMBED, _BATCH), jnp.float32),
    scratch_types=[
        pltpu.VMEM((_BATCH,), jnp.int32),
        pltpu.VMEM((_VOCAB,), jnp.float32),
        pltpu.VMEM((_BATCH,), jnp.float32),
        pltpu.SemaphoreType.DMA,
    ],
    compiler_params=pltpu.CompilerParams(needs_layout_passes=False),
)(_gather_body)


def _out_copy(acc_ref, out_hbm, sem_ref, step, last=False):
    ph = lax.rem(step, _NBUF)
    rows = _TAIL if last else _TV
    return pltpu.make_async_copy(
        acc_ref.at[ph, pl.ds(0, rows)],
        out_hbm.at[pl.ds(step * _TV, rows)],
        sem_ref.at[ph],
    )


def _proj_body(hidden_t_ref, wt_ref, out_hbm, acc_ref, sem_ref):
    i = pl.program_id(0)
    ph = lax.rem(i, _NBUF)

    @pl.when(i >= _NBUF)
    def _():
        _out_copy(acc_ref, out_hbm, sem_ref, i - _NBUF).wait()

    acc_ref[ph] = lax.dot_general(
        wt_ref[...],
        hidden_t_ref[...],
        (((0,), (0,)), ((), ())),
        preferred_element_type=jnp.float32,
    )

    @pl.when(i < _NV - 1)
    def _():
        _out_copy(acc_ref, out_hbm, sem_ref, i).start()

    @pl.when(i == _NV - 1)
    def _():
        _out_copy(acc_ref, out_hbm, sem_ref, i, last=True).start()
        for k in range(_NBUF - 1):
            _out_copy(acc_ref, out_hbm, sem_ref, _NV - _NBUF + k).wait()
        _out_copy(acc_ref, out_hbm, sem_ref, _NV - 1, last=True).wait()


def kernel(input, embed_table, expand_W):
    hidden_t = _gather(embed_table.T, input)
    logits_t = pl.pallas_call(
        _proj_body,
        grid=(_NV,),
        in_specs=[
            pl.BlockSpec((_EMBED, _BATCH), lambda i: (0, 0)),
            pl.BlockSpec((_EMBED, _TV), lambda i: (0, i)),
        ],
        out_specs=pl.BlockSpec(memory_space=pl.ANY),
        out_shape=jax.ShapeDtypeStruct((_VOCAB, _BATCH), jnp.float32),
        scratch_shapes=[
            pltpu.VMEM((_NBUF, _TV, _BATCH), jnp.float32),
            pltpu.SemaphoreType.DMA((_NBUF,)),
        ],
    )(hidden_t, expand_W.T)
    return logits_t.T
